# speedup vs baseline: 4.6111x; 4.6111x over previous
import functools
import math

import jax
import jax.numpy as jnp
from jax.experimental import pallas as pl
from jax.experimental.pallas import tpu as pltpu

_D = 1024
_H = _D // 2
_NA = 4
_SB = 512


def _body(x_ref, attr_ref, elem_ref, out_ref, pe_ref):
    j = pl.program_id(1)
    scale = math.sqrt(_D)

    @pl.when(j == 0)
    def _build_pe():
        rows = jax.lax.broadcasted_iota(jnp.int32, (_SB, _SB // _NA), 0)
        cols = jax.lax.broadcasted_iota(jnp.int32, (_SB, _SB // _NA), 1)
        sel_elem = (rows // _NA == cols).astype(jnp.float32)
        rows_a = jax.lax.broadcasted_iota(jnp.int32, (_SB, _NA), 0)
        cols_a = jax.lax.broadcasted_iota(jnp.int32, (_SB, _NA), 1)
        sel_attr = (rows_a % _NA == cols_a).astype(jnp.float32)
        pe_ref[:, :_H] = jnp.dot(sel_attr, attr_ref[...],
                                 preferred_element_type=jnp.float32)
        pe_ref[:, _H:] = jnp.dot(sel_elem, elem_ref[...],
                                 preferred_element_type=jnp.float32)

    out_ref[0] = x_ref[0] * scale + pe_ref[...]


def kernel(x, attr_embed, elem_embed):
    B, S, D = x.shape
    n_s = S // _SB
    grid = (n_s, B)
    return pl.pallas_call(
        _body,
        grid=grid,
        in_specs=[
            pl.BlockSpec((1, _SB, D), lambda i, j: (j, i, 0)),
            pl.BlockSpec((_NA, _H), lambda i, j: (0, 0)),
            pl.BlockSpec((_SB // _NA, _H), lambda i, j: (i, 0)),
        ],
        out_specs=pl.BlockSpec((1, _SB, D), lambda i, j: (j, i, 0)),
        out_shape=jax.ShapeDtypeStruct((B, S, D), jnp.float32),
        scratch_shapes=[pltpu.VMEM((_SB, D), jnp.float32)],
    )(x, attr_embed, elem_embed)
